# baseline (speedup 1.0000x reference)
import functools
import math

import jax
import jax.numpy as jnp
from jax import lax
from jax.experimental import pallas as pl
from jax.experimental.pallas import tpu as pltpu
from jax.experimental.pallas import tpu_sc as plsc

VOCAB = 100000
D = 768
B = 4
S = 4096
N = B * S
SCALE = math.sqrt(float(D))

_info = plsc.get_sparse_core_info()
NC = _info.num_cores
NS = _info.num_subcores
NW = NC * NS
ROWS_W = N // NW
R = 32
NCH = ROWS_W // R
LANES = 16
JV = D // LANES


def _sc_embed(x_flat, table, pe_s):
    mesh = plsc.VectorSubcoreMesh(core_axis_name="c", subcore_axis_name="s")

    @functools.partial(
        pl.kernel,
        mesh=mesh,
        out_type=jax.ShapeDtypeStruct((N, D), jnp.float32),
        scratch_types=[
            pltpu.VMEM((ROWS_W,), jnp.int32),
            pltpu.VMEM((R, D), jnp.float32),
            pltpu.VMEM((R, D), jnp.float32),
            pltpu.SemaphoreType.DMA,
        ],
    )
    def k(x_hbm, table_hbm, pe_hbm, out_hbm, idx_v, rows_v, pe_v, sem):
        wid = lax.axis_index("s") * NC + lax.axis_index("c")
        base = wid * ROWS_W
        s0 = (wid % (S // ROWS_W)) * ROWS_W
        pltpu.sync_copy(x_hbm.at[pl.ds(base, ROWS_W)], idx_v)

        def chunk(kc, _):
            off = kc * R
            gat = pltpu.async_copy(
                table_hbm.at[idx_v.at[pl.ds(off, R)]], rows_v, sem)
            pltpu.sync_copy(pe_hbm.at[pl.ds(s0 + off, R)], pe_v)
            gat.wait()

            def row(r, _):
                for j in range(JV):
                    sl = pl.ds(j * LANES, LANES)
                    rows_v[r, sl] = rows_v[r, sl] * SCALE + pe_v[r, sl]
                return 0

            lax.fori_loop(0, R, row, 0)
            pltpu.sync_copy(rows_v, out_hbm.at[pl.ds(base + off, R)])
            return 0

        lax.fori_loop(0, NCH, chunk, 0)

    return k(x_flat, table, pe_s)


def kernel(x, table, pe):
    out = _sc_embed(x.reshape(N), table, pe[:S])
    return out.reshape(B, S, D)
